# Initial kernel scaffold; baseline (speedup 1.0000x reference)
import functools

import jax
import jax.numpy as jnp
from jax import lax
from jax.experimental import pallas as pl
from jax.experimental.pallas import tpu as pltpu
from jax.experimental.pallas import tpu_sc as plsc

N = 50000
E = 800000
H = 64
HC = 2 * H
F = 32
NQ = 4
NC = 2
NS = 16

EPT = E // NS
B = 80
K = 25
G = B * K
NGRP = EPT // G
ROWS_PT = N // NS
WB = 625
NWB = ROWS_PT // WB

BR = 2000
GRID = N // BR


def _mesh():
  return plsc.VectorSubcoreMesh(
      core_axis_name="c", subcore_axis_name="s", num_cores=NC, num_subcores=NS)


@functools.partial(
    pl.kernel,
    out_type=jax.ShapeDtypeStruct((N, NQ, F), jnp.float32),
    mesh=_mesh(),
    scratch_types=[
        pltpu.VMEM_SHARED((N, F), jnp.float32),
        pltpu.VMEM((G,), jnp.int32),
        pltpu.VMEM((K, B), jnp.int32),
        pltpu.VMEM((K, B, F), jnp.float32),
        pltpu.VMEM((WB, F), jnp.float32),
        pltpu.VMEM((WB, F), jnp.float32),
        pltpu.SemaphoreType.DMA,
        pltpu.SemaphoreType.DMA,
    ],
)
def _sc_aggregate(t4_hbm, src_hbm, dst2_hbm, out_hbm,
                  acc, srcb, dstb, rowsb, zbuf, wbuf, semg, sems):
  cid = lax.axis_index("c")
  sid = lax.axis_index("s")
  ebase = sid * EPT
  d0 = sid * ROWS_PT

  def _zb(i, _):
    zbuf[i, pl.ds(0, 16)] = jnp.zeros((16,), jnp.float32)
    zbuf[i, pl.ds(16, 16)] = jnp.zeros((16,), jnp.float32)
    return _
  lax.fori_loop(0, WB, _zb, None)

  for p in range(2):
    q = 2 * cid + p

    for k in range(NWB):
      pltpu.sync_copy(zbuf, acc.at[pl.ds(d0 + k * WB, WB), :])
    plsc.subcore_barrier()

    def _group(g, _):
      base = ebase + g * G
      pltpu.sync_copy(src_hbm.at[pl.ds(base, G)], srcb)
      pltpu.sync_copy(dst2_hbm.at[pl.ds(base // B, K), :], dstb)

      def _adj(i, _):
        x = srcb[pl.ds(16 * i, 16)]
        srcb[pl.ds(16 * i, 16)] = x * 4 + q
        return _
      lax.fori_loop(0, G // 16, _adj, None)

      gathers = []
      for j in range(K):
        gathers.append(pltpu.async_copy(
            t4_hbm.at[srcb.at[pl.ds(j * B, B)]], rowsb.at[j], semg))
      for cp in gathers:
        cp.wait()
      scatters = []
      for j in range(K):
        scatters.append(pltpu.async_copy(
            rowsb.at[j], acc.at[dstb.at[j]], sems, add=True))
      for cp in scatters:
        cp.wait()
      return _
    lax.fori_loop(0, NGRP, _group, None)
    plsc.subcore_barrier()

    for k in range(NWB):
      pltpu.sync_copy(acc.at[pl.ds(d0 + k * WB, WB), :], wbuf)
      pltpu.sync_copy(wbuf, out_hbm.at[pl.ds(d0 + k * WB, WB), q, :])
    plsc.subcore_barrier()


def _tc_init_msg(s_ref, v_ref, wls, bls, wlv, blv, wblk, bcat, h_ref, t_ref):
  hs = jnp.maximum(jnp.dot(s_ref[...], wls[...],
                           preferred_element_type=jnp.float32) + bls[...], 0.0)
  hv = jnp.maximum(jnp.dot(v_ref[...], wlv[...],
                           preferred_element_type=jnp.float32) + blv[...], 0.0)
  h = jnp.concatenate([hs, hv], axis=1)
  h_ref[...] = h
  t_ref[...] = jnp.dot(h, wblk[...],
                       preferred_element_type=jnp.float32) + bcat[...]


def _tc_resid_msg(h_ref, agg_ref, wblk, bcat, h2_ref, t2_ref):
  h2 = jnp.maximum(h_ref[...] + agg_ref[...], 0.0)
  h2_ref[...] = h2
  t2_ref[...] = jnp.dot(h2, wblk[...],
                        preferred_element_type=jnp.float32) + bcat[...]


def _tc_final(h_ref, agg_ref, out_ref):
  out_ref[...] = jnp.maximum(h_ref[...] + agg_ref[...], 0.0)


def _row_block(width):
  return pl.BlockSpec((BR, width), lambda i: (i, 0))


def _full(shape):
  return pl.BlockSpec(shape, lambda i: tuple(0 for _ in shape))


def kernel(s, v, edge_index, W_lin_s, b_lin_s, W_lin_v, b_lin_v,
           W_msg_s, b_msg_s, W_msg_v, b_msg_v):
  src = edge_index[0].astype(jnp.int32)
  dst2 = edge_index[1].astype(jnp.int32).reshape(E // B, B)

  wblk = jnp.zeros((HC, HC), jnp.float32)
  wblk = wblk.at[:H, :H].set(W_msg_s).at[H:, H:].set(W_msg_v)
  bcat = jnp.concatenate([b_msg_s, b_msg_v]).reshape(1, HC)
  bls = b_lin_s.reshape(1, H)
  blv = b_lin_v.reshape(1, H)

  h, t = pl.pallas_call(
      _tc_init_msg,
      grid=(GRID,),
      in_specs=[_row_block(HC), _row_block(HC),
                _full((HC, H)), _full((1, H)), _full((HC, H)), _full((1, H)),
                _full((HC, HC)), _full((1, HC))],
      out_specs=[_row_block(HC), _row_block(HC)],
      out_shape=[jax.ShapeDtypeStruct((N, HC), jnp.float32),
                 jax.ShapeDtypeStruct((N, HC), jnp.float32)],
  )(s, v, W_lin_s, bls, W_lin_v, blv, wblk, bcat)

  out = None
  for it in range(2):
    agg = _sc_aggregate(t.reshape(NQ * N, F), src, dst2)
    agg = agg.reshape(N, HC)
    if it == 0:
      h, t = pl.pallas_call(
          _tc_resid_msg,
          grid=(GRID,),
          in_specs=[_row_block(HC), _row_block(HC),
                    _full((HC, HC)), _full((1, HC))],
          out_specs=[_row_block(HC), _row_block(HC)],
          out_shape=[jax.ShapeDtypeStruct((N, HC), jnp.float32),
                     jax.ShapeDtypeStruct((N, HC), jnp.float32)],
      )(h, agg, wblk, bcat)
    else:
      out = pl.pallas_call(
          _tc_final,
          grid=(GRID,),
          in_specs=[_row_block(HC), _row_block(HC)],
          out_specs=_row_block(HC),
          out_shape=jax.ShapeDtypeStruct((N, HC), jnp.float32),
      )(h, agg)
  return out


# speedup vs baseline: 7.5594x; 7.5594x over previous
import functools

import jax
import jax.numpy as jnp
from jax import lax
from jax.experimental import pallas as pl
from jax.experimental.pallas import tpu as pltpu
from jax.experimental.pallas import tpu_sc as plsc

N = 50000
E = 800000
H = 64
HC = 2 * H
F = 32
NQ = 4
NC = 2
NS = 16

EPT = E // NS
B = 80
K = 5
G = B * K
NGRP = EPT // G
ROWS_PT = N // NS
WB = 125
NWB = ROWS_PT // WB

BR = 2000
GRID = N // BR


def _mesh():
  return plsc.VectorSubcoreMesh(
      core_axis_name="c", subcore_axis_name="s", num_cores=NC, num_subcores=NS)


@functools.partial(
    pl.kernel,
    out_type=jax.ShapeDtypeStruct((N, NQ, F), jnp.float32),
    mesh=_mesh(),
    scratch_types=[
        pltpu.VMEM_SHARED((N, F), jnp.float32),
        pltpu.VMEM((G,), jnp.int32),
        pltpu.VMEM((K, B), jnp.int32),
        pltpu.VMEM((K, B, F), jnp.float32),
        pltpu.VMEM((WB, F), jnp.float32),
        pltpu.VMEM((WB, F), jnp.float32),
        pltpu.SemaphoreType.DMA,
        pltpu.SemaphoreType.DMA,
    ],
    compiler_params=pltpu.CompilerParams(use_tc_tiling_on_sc=False),
)
def _sc_aggregate(t4_hbm, src_hbm, dst2_hbm, out_hbm,
                  acc, srcb, dstb, rowsb, zbuf, wbuf, semg, sems):
  cid = lax.axis_index("c")
  sid = lax.axis_index("s")
  ebase = sid * EPT
  d0 = sid * ROWS_PT

  def _zb(i, _):
    zbuf[i, pl.ds(0, 16)] = jnp.zeros((16,), jnp.float32)
    zbuf[i, pl.ds(16, 16)] = jnp.zeros((16,), jnp.float32)
    return _
  lax.fori_loop(0, WB, _zb, None)

  for p in range(2):
    q = 2 * cid + p

    def _zero(k, _):
      pltpu.sync_copy(zbuf, acc.at[pl.ds(d0 + k * WB, WB), :])
      return _
    lax.fori_loop(0, NWB, _zero, None)
    plsc.subcore_barrier()

    def _group(g, _):
      base = ebase + g * G
      pltpu.sync_copy(src_hbm.at[pl.ds(base, G)], srcb)
      pltpu.sync_copy(dst2_hbm.at[pl.ds(base // B, K), :], dstb)

      def _adj(i, _):
        x = srcb[pl.ds(16 * i, 16)]
        srcb[pl.ds(16 * i, 16)] = x * 4 + q
        return _
      lax.fori_loop(0, G // 16, _adj, None)

      gathers = []
      for j in range(K):
        gathers.append(pltpu.async_copy(
            t4_hbm.at[srcb.at[pl.ds(j * B, B)]], rowsb.at[j], semg))
      for cp in gathers:
        cp.wait()
      scatters = []
      for j in range(K):
        scatters.append(pltpu.async_copy(
            rowsb.at[j], acc.at[dstb.at[j]], sems, add=True))
      for cp in scatters:
        cp.wait()
      return _
    lax.fori_loop(0, NGRP, _group, None)
    plsc.subcore_barrier()

    def _wb(k, _):
      pltpu.sync_copy(acc.at[pl.ds(d0 + k * WB, WB), :], wbuf)
      pltpu.sync_copy(wbuf, out_hbm.at[pl.ds(d0 + k * WB, WB), q, :])
      return _
    lax.fori_loop(0, NWB, _wb, None)
    plsc.subcore_barrier()


def _tc_init_msg(s_ref, v_ref, wls, bls, wlv, blv, wblk, bcat, h_ref, t_ref):
  hs = jnp.maximum(jnp.dot(s_ref[...], wls[...],
                           preferred_element_type=jnp.float32) + bls[...], 0.0)
  hv = jnp.maximum(jnp.dot(v_ref[...], wlv[...],
                           preferred_element_type=jnp.float32) + blv[...], 0.0)
  h = jnp.concatenate([hs, hv], axis=1)
  h_ref[...] = h
  t_ref[...] = jnp.dot(h, wblk[...],
                       preferred_element_type=jnp.float32) + bcat[...]


def _tc_resid_msg(h_ref, agg_ref, wblk, bcat, h2_ref, t2_ref):
  h2 = jnp.maximum(h_ref[...] + agg_ref[...], 0.0)
  h2_ref[...] = h2
  t2_ref[...] = jnp.dot(h2, wblk[...],
                        preferred_element_type=jnp.float32) + bcat[...]


def _tc_final(h_ref, agg_ref, out_ref):
  out_ref[...] = jnp.maximum(h_ref[...] + agg_ref[...], 0.0)


def _row_block(width):
  return pl.BlockSpec((BR, width), lambda i: (i, 0))


def _full(shape):
  return pl.BlockSpec(shape, lambda i: tuple(0 for _ in shape))


def kernel(s, v, edge_index, W_lin_s, b_lin_s, W_lin_v, b_lin_v,
           W_msg_s, b_msg_s, W_msg_v, b_msg_v):
  src = edge_index[0].astype(jnp.int32)
  dst2 = edge_index[1].astype(jnp.int32).reshape(E // B, B)

  wblk = jnp.zeros((HC, HC), jnp.float32)
  wblk = wblk.at[:H, :H].set(W_msg_s).at[H:, H:].set(W_msg_v)
  bcat = jnp.concatenate([b_msg_s, b_msg_v]).reshape(1, HC)
  bls = b_lin_s.reshape(1, H)
  blv = b_lin_v.reshape(1, H)

  h, t = pl.pallas_call(
      _tc_init_msg,
      grid=(GRID,),
      in_specs=[_row_block(HC), _row_block(HC),
                _full((HC, H)), _full((1, H)), _full((HC, H)), _full((1, H)),
                _full((HC, HC)), _full((1, HC))],
      out_specs=[_row_block(HC), _row_block(HC)],
      out_shape=[jax.ShapeDtypeStruct((N, HC), jnp.float32),
                 jax.ShapeDtypeStruct((N, HC), jnp.float32)],
  )(s, v, W_lin_s, bls, W_lin_v, blv, wblk, bcat)

  out = None
  for it in range(2):
    agg = _sc_aggregate(t.reshape(NQ * N, F), src, dst2)
    agg = agg.reshape(N, HC)
    if it == 0:
      h, t = pl.pallas_call(
          _tc_resid_msg,
          grid=(GRID,),
          in_specs=[_row_block(HC), _row_block(HC),
                    _full((HC, HC)), _full((1, HC))],
          out_specs=[_row_block(HC), _row_block(HC)],
          out_shape=[jax.ShapeDtypeStruct((N, HC), jnp.float32),
                     jax.ShapeDtypeStruct((N, HC), jnp.float32)],
      )(h, agg, wblk, bcat)
    else:
      out = pl.pallas_call(
          _tc_final,
          grid=(GRID,),
          in_specs=[_row_block(HC), _row_block(HC)],
          out_specs=_row_block(HC),
          out_shape=jax.ShapeDtypeStruct((N, HC), jnp.float32),
      )(h, agg)
  return out
